# baseline (speedup 1.0000x reference)
import jax
import jax.numpy as jnp
from jax import lax
from jax.experimental import pallas as pl
from jax.experimental.pallas import tpu as pltpu, tpu_sc as plsc

EMBED = 768
NEXP = 8
K = 384
INTER_S = 1536
T = 2048
TB = 256
NW = 32
TOK_PER_W = T // NW
CHT = 8
NCHUNK = TOK_PER_W // CHT
NV = EMBED // 16
KV = K // 16
MASK31 = 0x7FFFFFFF

_CONTRACT_MINOR = (((1,), (1,)), ((), ()))


def _dotT(a, b):
    return lax.dot_general(a, b, _CONTRACT_MINOR,
                           preferred_element_type=jnp.float32)


def _pe_body(pos_ref, Wp_ref, bp_ref, pe_ref):
    logits = _dotT(pos_ref[...], Wp_ref[...]) + bp_ref[...]
    z = logits - jnp.max(logits, axis=-1, keepdims=True)
    e = jnp.exp(z)
    pe_ref[...] = e / jnp.sum(e, axis=-1, keepdims=True)


def _gate_body(h_ref, pe_ref, Wg_ref, bg_ref, gate_ref):
    h = h_ref[...]
    for e in range(NEXP):
        gate_ref[e] = _dotT(h * pe_ref[e], Wg_ref[...]) + bg_ref[...]


def _shared_body(h_ref, Wgs_ref, Wus_ref, Wds_ref, y0_ref):
    h = h_ref[...]
    m = jax.nn.silu(_dotT(h, Wgs_ref[...])) * _dotT(h, Wus_ref[...])
    y0_ref[...] = _dotT(m, Wds_ref[...])


def _post_body(xe_ref, y0_ref, Wge_ref, Wue_ref, Wde_ref,
               lng_ref, lnb_ref, W1_ref, b1_ref, W2_ref, b2_ref, o_ref):
    y = y0_ref[...]
    for e in range(NEXP):
        x = xe_ref[e]
        a = _dotT(x, Wge_ref[e])
        b = _dotT(x, Wue_ref[e])
        y = y + _dotT(jax.nn.silu(a) * b, Wde_ref[e])
    mean = jnp.mean(y, axis=-1, keepdims=True)
    var = jnp.mean((y - mean) ** 2, axis=-1, keepdims=True)
    y = (y - mean) * lax.rsqrt(var + 1e-6) * lng_ref[...] + lnb_ref[...]
    z = jax.nn.silu(_dotT(y, W1_ref[...]) + b1_ref[...])
    o_ref[...] = _dotT(z, W2_ref[...]) + b2_ref[...]


def _desc_key(f):
    u = plsc.bitcast(f, jnp.int32)
    s = lax.shift_right_arithmetic(u, 31)
    return jnp.bitwise_xor(u, jnp.bitwise_and(jnp.bitwise_not(s), MASK31))


def _inv_desc_key(k):
    s = lax.shift_right_arithmetic(k, 31)
    u = jnp.bitwise_xor(k, jnp.bitwise_and(jnp.bitwise_not(s), MASK31))
    return plsc.bitcast(u, jnp.float32)


def _digit(k, p):
    if p:
        k = lax.shift_right_logical(k, 8 * p)
    return jnp.bitwise_and(k, 255)


def _sc_body(gate_hbm, h_hbm, xe_hbm,
             hbuf, gbuf, xbuf,
             kA0, kB0, pA0, pB0, ebuf0, h40, offs0, sbuf0, dbuf0, cbuf0, mbuf0,
             kA1, kB1, pA1, pB1, ebuf1, h41, offs1, sbuf1, dbuf1, cbuf1, mbuf1):
    wid = lax.axis_index("s") * 2 + lax.axis_index("c")
    zero16 = jnp.zeros((16,), jnp.int32)
    full15 = jnp.full((16,), 15, jnp.int32)
    C0 = dict(kA=kA0, kB=kB0, pA=pA0, pB=pB0, ebuf=ebuf0, h4=h40,
              offs=offs0, sbuf=sbuf0, dbuf=dbuf0, cbuf=cbuf0, mbuf=mbuf0)
    C1 = dict(kA=kA1, kB=kB1, pA=pA1, pB=pB1, ebuf=ebuf1, h4=h41,
              offs=offs1, sbuf=sbuf1, dbuf=dbuf1, cbuf=cbuf1, mbuf=mbuf1)
    CC = (C0, C1)

    def scan_offsets(p):
        run = [jnp.int32(-1), jnp.int32(-1)]
        for v in range(16):
            sl = pl.ds(v * 16, 16)
            slh = pl.ds(p * 256 + v * 16, 16)
            for i, c in enumerate(CC):
                hv = c["h4"][slh]
                inc = plsc.cumsum(hv)
                c["offs"][sl] = inc - hv + run[i]
                run[i] = run[i] + jnp.sum(hv)

    def radix_pass(p, rr):
        ks = "kA" if p % 2 == 0 else "kB"
        kd = "kB" if p % 2 == 0 else "kA"
        pd = "pB" if p % 2 == 0 else "pA"

        def psrc(c, i, sl):
            if p == 0:
                return hbuf[rr[i], sl]
            return c["pB" if p % 2 == 1 else "pA"][sl]

        scan_offsets(p)
        if p > 0:
            for v in range(NV):
                sl = pl.ds(v * 16, 16)
                for c in CC:
                    d = _digit(c[ks][sl], p)
                    cnt, lastm = plsc.scan_count(d)
                    c["dbuf"][sl] = d
                    c["cbuf"][sl] = cnt
                    c["mbuf"][sl] = jnp.where(lastm, 1, 0)
        for v in range(NV):
            sl = pl.ds(v * 16, 16)
            for i, c in enumerate(CC):
                d = c["dbuf"][sl]
                cnt = c["cbuf"][sl]
                lastm = c["mbuf"][sl] == 1
                base = plsc.load_gather(c["offs"], [d])
                pos = base + cnt
                plsc.store_scatter(c[kd], [pos], c[ks][sl])
                plsc.store_scatter(c[pd], [pos], psrc(c, i, sl))
                plsc.addupdate_scatter(c["offs"], [d], cnt, mask=lastm)

    def row_pair_body(rp, carry):
        rr = (2 * rp, 2 * rp + 1)
        for v in range(64):
            sl = pl.ds(v * 16, 16)
            for c in CC:
                c["h4"][sl] = zero16
        for v in range(NV):
            sl = pl.ds(v * 16, 16)
            for i, c in enumerate(CC):
                k = _desc_key(gbuf[rr[i], sl])
                c["kA"][sl] = k
                d0 = _digit(k, 0)
                d1 = _digit(k, 1)
                d2 = _digit(k, 2)
                d3 = _digit(k, 3)
                cnt0, last0 = plsc.scan_count(d0)
                cnt1, last1 = plsc.scan_count(d1)
                cnt2, last2 = plsc.scan_count(d2)
                cnt3, last3 = plsc.scan_count(d3)
                plsc.addupdate_scatter(c["h4"], [d0], cnt0, mask=last0)
                plsc.addupdate_scatter(c["h4"], [d1 + 256], cnt1, mask=last1)
                plsc.addupdate_scatter(c["h4"], [d2 + 512], cnt2, mask=last2)
                plsc.addupdate_scatter(c["h4"], [d3 + 768], cnt3, mask=last3)
                c["dbuf"][sl] = d0
                c["cbuf"][sl] = cnt0
                c["mbuf"][sl] = jnp.where(last0, 1, 0)
        for p in range(4):
            radix_pass(p, rr)
        acc = [jnp.zeros((16,), jnp.float32)] * 2
        mv = [None, None]
        for i, c in enumerate(CC):
            v0 = _inv_desc_key(c["kA"][pl.ds(0, 16)])
            c["sbuf"][...] = plsc.cummax(v0)
            mv[i] = plsc.load_gather(c["sbuf"], [full15])
        for j in range(KV):
            sl = pl.ds(j * 16, 16)
            for i, c in enumerate(CC):
                ej = jnp.exp(_inv_desc_key(c["kA"][sl]) - mv[i])
                c["ebuf"][sl] = ej
                acc[i] = acc[i] + ej
        winv = [None, None]
        for i, c in enumerate(CC):
            c["sbuf"][...] = plsc.cumsum(acc[i])
            tot = plsc.load_gather(c["sbuf"], [full15])
            winv[i] = 1.0 / tot
        for j in range(KV):
            sl = pl.ds(j * 16, 16)
            for i, c in enumerate(CC):
                xbuf[rr[i], sl] = c["ebuf"][sl] * winv[i] * c["pA"][sl]
        return carry

    def exp_body(e, carry, t0):
        pltpu.sync_copy(gate_hbm.at[e, pl.ds(t0, CHT)], gbuf)
        lax.fori_loop(0, CHT // 2, row_pair_body, 0)
        pltpu.sync_copy(xbuf, xe_hbm.at[e, pl.ds(t0, CHT)])
        return carry

    def chunk_body(ci, carry):
        t0 = wid * TOK_PER_W + ci * CHT
        pltpu.sync_copy(h_hbm.at[pl.ds(t0, CHT)], hbuf)
        lax.fori_loop(0, NEXP, lambda e, c: exp_body(e, c, t0), 0)
        return carry

    lax.fori_loop(0, NCHUNK, chunk_body, 0)


def _sc_topk(gate_all, h):
    mesh = plsc.VectorSubcoreMesh(core_axis_name="c", subcore_axis_name="s")
    return pl.kernel(
        _sc_body,
        out_type=jax.ShapeDtypeStruct((NEXP, T, K), jnp.float32),
        mesh=mesh,
        compiler_params=pltpu.CompilerParams(needs_layout_passes=False),
        scratch_types=[
            pltpu.VMEM((CHT, EMBED), jnp.float32),
            pltpu.VMEM((CHT, EMBED), jnp.float32),
            pltpu.VMEM((CHT, K), jnp.float32),
        ] + 2 * [
            pltpu.VMEM((EMBED,), jnp.int32),
            pltpu.VMEM((EMBED,), jnp.int32),
            pltpu.VMEM((EMBED,), jnp.float32),
            pltpu.VMEM((EMBED,), jnp.float32),
            pltpu.VMEM((K,), jnp.float32),
            pltpu.VMEM((1024,), jnp.int32),
            pltpu.VMEM((256,), jnp.int32),
            pltpu.VMEM((16,), jnp.float32),
            pltpu.VMEM((EMBED,), jnp.int32),
            pltpu.VMEM((EMBED,), jnp.int32),
            pltpu.VMEM((EMBED,), jnp.int32),
        ],
    )(gate_all, h)


def kernel(hidden_states, posembed, W_pos, b_pos, W_gate, b_gate,
           Wg_e, Wu_e, Wd_e, Wg_s, Wu_s, Wd_s,
           ln_g, ln_b, W1, b1, W2, b2):
    h = hidden_states[0]
    b_pos2 = b_pos[None]
    b_gate2 = b_gate[None]

    pe = pl.pallas_call(
        _pe_body,
        out_shape=jax.ShapeDtypeStruct((NEXP, EMBED), jnp.float32),
    )(posembed, W_pos, b_pos2)

    grid = (T // TB,)
    full2 = lambda a, b: pl.BlockSpec((a, b), lambda i: (0, 0))
    tile2 = lambda w: pl.BlockSpec((TB, w), lambda i: (i, 0))

    gate_all = pl.pallas_call(
        _gate_body,
        grid=grid,
        in_specs=[tile2(EMBED), full2(NEXP, EMBED), full2(EMBED, EMBED),
                  full2(1, EMBED)],
        out_specs=pl.BlockSpec((NEXP, TB, EMBED), lambda i: (0, i, 0)),
        out_shape=jax.ShapeDtypeStruct((NEXP, T, EMBED), jnp.float32),
    )(h, pe, W_gate, b_gate2)

    xe = _sc_topk(gate_all, h)

    y0 = pl.pallas_call(
        _shared_body,
        grid=grid,
        in_specs=[tile2(EMBED), full2(INTER_S, EMBED), full2(INTER_S, EMBED),
                  full2(EMBED, INTER_S)],
        out_specs=tile2(EMBED),
        out_shape=jax.ShapeDtypeStruct((T, EMBED), jnp.float32),
    )(h, Wg_s, Wu_s, Wd_s)

    full3 = lambda s: pl.BlockSpec(s, lambda i: (0, 0, 0))
    out = pl.pallas_call(
        _post_body,
        grid=grid,
        in_specs=[pl.BlockSpec((NEXP, TB, K), lambda i: (0, i, 0)),
                  tile2(EMBED),
                  full3((NEXP, EMBED, K)), full3((NEXP, EMBED, K)),
                  full3((NEXP, EMBED, EMBED)),
                  full2(1, EMBED), full2(1, EMBED),
                  full2(EMBED, EMBED), full2(1, EMBED),
                  full2(EMBED, EMBED), full2(1, EMBED)],
        out_specs=tile2(EMBED),
        out_shape=jax.ShapeDtypeStruct((T, EMBED), jnp.float32),
    )(xe, y0, Wg_e, Wu_e, Wd_e, ln_g[None], ln_b[None],
      W1, b1[None], W2, b2[None])
    return out[None]


# speedup vs baseline: 1.0638x; 1.0638x over previous
import jax
import jax.numpy as jnp
from jax import lax
from jax.experimental import pallas as pl
from jax.experimental.pallas import tpu as pltpu, tpu_sc as plsc

EMBED = 768
NEXP = 8
K = 384
INTER_S = 1536
T = 2048
TB = 256
NW = 32
TOK_PER_W = T // NW
CHT = 8
NCHUNK = TOK_PER_W // CHT
NV = EMBED // 16
KV = K // 16
MASK31 = 0x7FFFFFFF

_CONTRACT_MINOR = (((1,), (1,)), ((), ()))


def _dotT(a, b):
    return lax.dot_general(a, b, _CONTRACT_MINOR,
                           preferred_element_type=jnp.float32)


def _pe_body(pos_ref, Wp_ref, bp_ref, pe_ref):
    logits = _dotT(pos_ref[...], Wp_ref[...]) + bp_ref[...]
    z = logits - jnp.max(logits, axis=-1, keepdims=True)
    e = jnp.exp(z)
    pe_ref[...] = e / jnp.sum(e, axis=-1, keepdims=True)


def _gate_body(h_ref, pe_ref, Wg_ref, bg_ref, gate_ref):
    h = h_ref[...]
    for e in range(NEXP):
        gate_ref[e] = _dotT(h * pe_ref[e], Wg_ref[...]) + bg_ref[...]


def _shared_body(h_ref, Wgs_ref, Wus_ref, Wds_ref, y0_ref):
    h = h_ref[...]
    m = jax.nn.silu(_dotT(h, Wgs_ref[...])) * _dotT(h, Wus_ref[...])
    y0_ref[...] = _dotT(m, Wds_ref[...])


def _post_body(xe_ref, y0_ref, Wge_ref, Wue_ref, Wde_ref,
               lng_ref, lnb_ref, W1_ref, b1_ref, W2_ref, b2_ref, o_ref):
    y = y0_ref[...]
    for e in range(NEXP):
        x = xe_ref[e]
        a = _dotT(x, Wge_ref[e])
        b = _dotT(x, Wue_ref[e])
        y = y + _dotT(jax.nn.silu(a) * b, Wde_ref[e])
    mean = jnp.mean(y, axis=-1, keepdims=True)
    var = jnp.mean((y - mean) ** 2, axis=-1, keepdims=True)
    y = (y - mean) * lax.rsqrt(var + 1e-6) * lng_ref[...] + lnb_ref[...]
    z = jax.nn.silu(_dotT(y, W1_ref[...]) + b1_ref[...])
    o_ref[...] = _dotT(z, W2_ref[...]) + b2_ref[...]


def _desc_key(f):
    u = plsc.bitcast(f, jnp.int32)
    s = lax.shift_right_arithmetic(u, 31)
    return jnp.bitwise_xor(u, jnp.bitwise_and(jnp.bitwise_not(s), MASK31))


def _inv_desc_key(k):
    s = lax.shift_right_arithmetic(k, 31)
    u = jnp.bitwise_xor(k, jnp.bitwise_and(jnp.bitwise_not(s), MASK31))
    return plsc.bitcast(u, jnp.float32)


def _digit(k, p):
    if p:
        k = lax.shift_right_logical(k, 8 * p)
    return jnp.bitwise_and(k, 255)


NCTX = 2
_CTX_BUFS = ("kA", "kB", "pA", "pB", "ebuf", "h4", "offs", "sbuf")


def _sc_body(gate_hbm, h_hbm, xe_hbm, hbuf, gbuf, xbuf, *ctx_bufs):
    wid = lax.axis_index("s") * 2 + lax.axis_index("c")
    zero16 = jnp.zeros((16,), jnp.int32)
    full15 = jnp.full((16,), 15, jnp.int32)
    nb = len(_CTX_BUFS)
    CC = tuple(dict(zip(_CTX_BUFS, ctx_bufs[i * nb:(i + 1) * nb]))
               for i in range(NCTX))

    def scan_offsets(p):
        run = [jnp.int32(-1)] * NCTX
        for v in range(16):
            sl = pl.ds(v * 16, 16)
            slh = pl.ds(p * 256 + v * 16, 16)
            for i, c in enumerate(CC):
                hv = c["h4"][slh]
                inc = plsc.cumsum(hv)
                c["offs"][sl] = inc - hv + run[i]
                run[i] = run[i] + jnp.sum(hv)

    def radix_pass(p, rr):
        ks = "kA" if p % 2 == 0 else "kB"
        kd = "kB" if p % 2 == 0 else "kA"
        pd = "pB" if p % 2 == 0 else "pA"

        def psrc(c, i, sl):
            if p == 0:
                return hbuf[rr[i], sl]
            return c["pB" if p % 2 == 1 else "pA"][sl]

        scan_offsets(p)
        for v in range(NV):
            sl = pl.ds(v * 16, 16)
            for i, c in enumerate(CC):
                k = c[ks][sl]
                d = _digit(k, p)
                cnt, lastm = plsc.scan_count(d)
                base = plsc.load_gather(c["offs"], [d])
                pos = base + cnt
                plsc.store_scatter(c[kd], [pos], k)
                plsc.store_scatter(c[pd], [pos], psrc(c, i, sl))
                plsc.addupdate_scatter(c["offs"], [d], cnt, mask=lastm)

    def row_group_body(rp, carry):
        rr = tuple(NCTX * rp + i for i in range(NCTX))
        for v in range(64):
            sl = pl.ds(v * 16, 16)
            for c in CC:
                c["h4"][sl] = zero16
        for v in range(NV):
            sl = pl.ds(v * 16, 16)
            for i, c in enumerate(CC):
                k = _desc_key(gbuf[rr[i], sl])
                c["kA"][sl] = k
                d0 = _digit(k, 0)
                d1 = _digit(k, 1)
                d2 = _digit(k, 2)
                d3 = _digit(k, 3)
                cnt0, last0 = plsc.scan_count(d0)
                cnt1, last1 = plsc.scan_count(d1)
                cnt2, last2 = plsc.scan_count(d2)
                cnt3, last3 = plsc.scan_count(d3)
                plsc.addupdate_scatter(c["h4"], [d0], cnt0, mask=last0)
                plsc.addupdate_scatter(c["h4"], [d1 + 256], cnt1, mask=last1)
                plsc.addupdate_scatter(c["h4"], [d2 + 512], cnt2, mask=last2)
                plsc.addupdate_scatter(c["h4"], [d3 + 768], cnt3, mask=last3)
        for p in range(4):
            radix_pass(p, rr)
        acc = [jnp.zeros((16,), jnp.float32)] * NCTX
        mv = [None] * NCTX
        for i, c in enumerate(CC):
            v0 = _inv_desc_key(c["kA"][pl.ds(0, 16)])
            c["sbuf"][...] = plsc.cummax(v0)
            mv[i] = plsc.load_gather(c["sbuf"], [full15])
        for j in range(KV):
            sl = pl.ds(j * 16, 16)
            for i, c in enumerate(CC):
                ej = jnp.exp(_inv_desc_key(c["kA"][sl]) - mv[i])
                c["ebuf"][sl] = ej
                acc[i] = acc[i] + ej
        winv = [None] * NCTX
        for i, c in enumerate(CC):
            c["sbuf"][...] = plsc.cumsum(acc[i])
            tot = plsc.load_gather(c["sbuf"], [full15])
            winv[i] = 1.0 / tot
        for j in range(KV):
            sl = pl.ds(j * 16, 16)
            for i, c in enumerate(CC):
                xbuf[rr[i], sl] = c["ebuf"][sl] * winv[i] * c["pA"][sl]
        return carry

    def exp_body(e, carry, t0):
        pltpu.sync_copy(gate_hbm.at[e, pl.ds(t0, CHT)], gbuf)
        lax.fori_loop(0, CHT // NCTX, row_group_body, 0)
        pltpu.sync_copy(xbuf, xe_hbm.at[e, pl.ds(t0, CHT)])
        return carry

    def chunk_body(ci, carry):
        t0 = wid * TOK_PER_W + ci * CHT
        pltpu.sync_copy(h_hbm.at[pl.ds(t0, CHT)], hbuf)
        lax.fori_loop(0, NEXP, lambda e, c: exp_body(e, c, t0), 0)
        return carry

    lax.fori_loop(0, NCHUNK, chunk_body, 0)


def _sc_topk(gate_all, h):
    mesh = plsc.VectorSubcoreMesh(core_axis_name="c", subcore_axis_name="s")
    return pl.kernel(
        _sc_body,
        out_type=jax.ShapeDtypeStruct((NEXP, T, K), jnp.float32),
        mesh=mesh,
        compiler_params=pltpu.CompilerParams(needs_layout_passes=False),
        scratch_types=[
            pltpu.VMEM((CHT, EMBED), jnp.float32),
            pltpu.VMEM((CHT, EMBED), jnp.float32),
            pltpu.VMEM((CHT, K), jnp.float32),
        ] + NCTX * [
            pltpu.VMEM((EMBED,), jnp.int32),
            pltpu.VMEM((EMBED,), jnp.int32),
            pltpu.VMEM((EMBED,), jnp.float32),
            pltpu.VMEM((EMBED,), jnp.float32),
            pltpu.VMEM((K,), jnp.float32),
            pltpu.VMEM((1024,), jnp.int32),
            pltpu.VMEM((256,), jnp.int32),
            pltpu.VMEM((16,), jnp.float32),
        ],
    )(gate_all, h)


def kernel(hidden_states, posembed, W_pos, b_pos, W_gate, b_gate,
           Wg_e, Wu_e, Wd_e, Wg_s, Wu_s, Wd_s,
           ln_g, ln_b, W1, b1, W2, b2):
    h = hidden_states[0]
    b_pos2 = b_pos[None]
    b_gate2 = b_gate[None]

    pe = pl.pallas_call(
        _pe_body,
        out_shape=jax.ShapeDtypeStruct((NEXP, EMBED), jnp.float32),
    )(posembed, W_pos, b_pos2)

    grid = (T // TB,)
    full2 = lambda a, b: pl.BlockSpec((a, b), lambda i: (0, 0))
    tile2 = lambda w: pl.BlockSpec((TB, w), lambda i: (i, 0))

    gate_all = pl.pallas_call(
        _gate_body,
        grid=grid,
        in_specs=[tile2(EMBED), full2(NEXP, EMBED), full2(EMBED, EMBED),
                  full2(1, EMBED)],
        out_specs=pl.BlockSpec((NEXP, TB, EMBED), lambda i: (0, i, 0)),
        out_shape=jax.ShapeDtypeStruct((NEXP, T, EMBED), jnp.float32),
    )(h, pe, W_gate, b_gate2)

    xe = _sc_topk(gate_all, h)

    y0 = pl.pallas_call(
        _shared_body,
        grid=grid,
        in_specs=[tile2(EMBED), full2(INTER_S, EMBED), full2(INTER_S, EMBED),
                  full2(EMBED, INTER_S)],
        out_specs=tile2(EMBED),
        out_shape=jax.ShapeDtypeStruct((T, EMBED), jnp.float32),
    )(h, Wg_s, Wu_s, Wd_s)

    full3 = lambda s: pl.BlockSpec(s, lambda i: (0, 0, 0))
    out = pl.pallas_call(
        _post_body,
        grid=grid,
        in_specs=[pl.BlockSpec((NEXP, TB, K), lambda i: (0, i, 0)),
                  tile2(EMBED),
                  full3((NEXP, EMBED, K)), full3((NEXP, EMBED, K)),
                  full3((NEXP, EMBED, EMBED)),
                  full2(1, EMBED), full2(1, EMBED),
                  full2(EMBED, EMBED), full2(1, EMBED),
                  full2(EMBED, EMBED), full2(1, EMBED)],
        out_specs=tile2(EMBED),
        out_shape=jax.ShapeDtypeStruct((T, EMBED), jnp.float32),
    )(xe, y0, Wg_e, Wu_e, Wd_e, ln_g[None], ln_b[None],
      W1, b1[None], W2, b2[None])
    return out[None]


# speedup vs baseline: 1.6251x; 1.5276x over previous
import jax
import jax.numpy as jnp
from jax import lax
from jax.experimental import pallas as pl
from jax.experimental.pallas import tpu as pltpu, tpu_sc as plsc

EMBED = 768
NEXP = 8
K = 384
INTER_S = 1536
T = 2048
TB = 256
NW = 32
TOK_PER_W = T // NW
CHT = 8
NCHUNK = TOK_PER_W // CHT
NV = EMBED // 16
KV = K // 16
MASK31 = 0x7FFFFFFF

_CONTRACT_MINOR = (((1,), (1,)), ((), ()))


def _dotT(a, b):
    return lax.dot_general(a, b, _CONTRACT_MINOR,
                           preferred_element_type=jnp.float32)


def _pe_body(pos_ref, Wp_ref, bp_ref, pe_ref):
    logits = _dotT(pos_ref[...], Wp_ref[...]) + bp_ref[...]
    z = logits - jnp.max(logits, axis=-1, keepdims=True)
    e = jnp.exp(z)
    pe_ref[...] = e / jnp.sum(e, axis=-1, keepdims=True)


def _gate_body(h_ref, pe_ref, Wg_ref, bg_ref, gate_ref):
    h = h_ref[...]
    for e in range(NEXP):
        gate_ref[e] = _dotT(h * pe_ref[e], Wg_ref[...]) + bg_ref[...]


def _shared_body(h_ref, Wgs_ref, Wus_ref, Wds_ref, y0_ref):
    h = h_ref[...]
    m = jax.nn.silu(_dotT(h, Wgs_ref[...])) * _dotT(h, Wus_ref[...])
    y0_ref[...] = _dotT(m, Wds_ref[...])


def _post_body(xe_ref, y0_ref, Wge_ref, Wue_ref, Wde_ref,
               lng_ref, lnb_ref, W1_ref, b1_ref, W2_ref, b2_ref, o_ref):
    y = y0_ref[...]
    for e in range(NEXP):
        x = xe_ref[e]
        a = _dotT(x, Wge_ref[e])
        b = _dotT(x, Wue_ref[e])
        y = y + _dotT(jax.nn.silu(a) * b, Wde_ref[e])
    mean = jnp.mean(y, axis=-1, keepdims=True)
    var = jnp.mean((y - mean) ** 2, axis=-1, keepdims=True)
    y = (y - mean) * lax.rsqrt(var + 1e-6) * lng_ref[...] + lnb_ref[...]
    z = jax.nn.silu(_dotT(y, W1_ref[...]) + b1_ref[...])
    o_ref[...] = _dotT(z, W2_ref[...]) + b2_ref[...]


def _desc_key(f):
    u = plsc.bitcast(f, jnp.int32)
    s = lax.shift_right_arithmetic(u, 31)
    return jnp.bitwise_xor(u, jnp.bitwise_and(jnp.bitwise_not(s), MASK31))


def _inv_desc_key(k):
    s = lax.shift_right_arithmetic(k, 31)
    u = jnp.bitwise_xor(k, jnp.bitwise_and(jnp.bitwise_not(s), MASK31))
    return plsc.bitcast(u, jnp.float32)


def _digit(k, p):
    if p:
        k = lax.shift_right_logical(k, 8 * p)
    return jnp.bitwise_and(k, 255)


NCTX = 4
UF = 4
_CTX_BUFS = ("kA", "kB", "pA", "pB", "ebuf", "h4", "offs", "sbuf")


def _sc_body(gate_hbm, h_hbm, xe_hbm, hbuf, gbuf, xbuf, *ctx_bufs):
    wid = lax.axis_index("s") * 2 + lax.axis_index("c")
    zero16 = jnp.zeros((16,), jnp.int32)
    full15 = jnp.full((16,), 15, jnp.int32)
    nb = len(_CTX_BUFS)
    CC = tuple(dict(zip(_CTX_BUFS, ctx_bufs[i * nb:(i + 1) * nb]))
               for i in range(NCTX))

    def scan_offsets(p):
        run = [jnp.int32(-1)] * NCTX
        for v in range(16):
            sl = pl.ds(v * 16, 16)
            slh = pl.ds(p * 256 + v * 16, 16)
            for i, c in enumerate(CC):
                hv = c["h4"][slh]
                inc = plsc.cumsum(hv)
                c["offs"][sl] = inc - hv + run[i]
                run[i] = run[i] + jnp.sum(hv)

    def vloop(n_vregs, body):
        def it_body(it, carry):
            for j in range(UF):
                body(it * UF + j)
            return carry
        lax.fori_loop(0, n_vregs // UF, it_body, 0)

    def radix_pass(p, rr):
        ks = "kA" if p % 2 == 0 else "kB"
        kd = "kB" if p % 2 == 0 else "kA"
        pd = "pB" if p % 2 == 0 else "pA"

        def psrc(c, i, sl):
            if p == 0:
                return hbuf[rr[i], sl]
            return c["pB" if p % 2 == 1 else "pA"][sl]

        scan_offsets(p)

        def body(v):
            sl = pl.ds(v * 16, 16)
            for i, c in enumerate(CC):
                k = c[ks][sl]
                d = _digit(k, p)
                cnt, lastm = plsc.scan_count(d)
                base = plsc.load_gather(c["offs"], [d])
                pos = base + cnt
                plsc.store_scatter(c[kd], [pos], k)
                plsc.store_scatter(c[pd], [pos], psrc(c, i, sl))
                plsc.addupdate_scatter(c["offs"], [d], cnt, mask=lastm)
        vloop(NV, body)

    def row_group_body(rp, carry):
        rr = tuple(NCTX * rp + i for i in range(NCTX))

        def zero_body(v):
            sl = pl.ds(v * 16, 16)
            for c in CC:
                c["h4"][sl] = zero16
        vloop(64, zero_body)

        def hist_body(v):
            sl = pl.ds(v * 16, 16)
            for i, c in enumerate(CC):
                k = _desc_key(gbuf[rr[i], sl])
                c["kA"][sl] = k
                d0 = _digit(k, 0)
                d1 = _digit(k, 1)
                d2 = _digit(k, 2)
                d3 = _digit(k, 3)
                cnt0, last0 = plsc.scan_count(d0)
                cnt1, last1 = plsc.scan_count(d1)
                cnt2, last2 = plsc.scan_count(d2)
                cnt3, last3 = plsc.scan_count(d3)
                plsc.addupdate_scatter(c["h4"], [d0], cnt0, mask=last0)
                plsc.addupdate_scatter(c["h4"], [d1 + 256], cnt1, mask=last1)
                plsc.addupdate_scatter(c["h4"], [d2 + 512], cnt2, mask=last2)
                plsc.addupdate_scatter(c["h4"], [d3 + 768], cnt3, mask=last3)
        vloop(NV, hist_body)

        for p in range(4):
            radix_pass(p, rr)
        mv = [None] * NCTX
        for i, c in enumerate(CC):
            v0 = _inv_desc_key(c["kA"][pl.ds(0, 16)])
            c["sbuf"][...] = plsc.cummax(v0)
            mv[i] = plsc.load_gather(c["sbuf"], [full15])

        def exp_it(it, accs):
            new = list(accs)
            for j in range(UF):
                sl = pl.ds((it * UF + j) * 16, 16)
                for i, c in enumerate(CC):
                    ej = jnp.exp(_inv_desc_key(c["kA"][sl]) - mv[i])
                    c["ebuf"][sl] = ej
                    new[i] = new[i] + ej
            return tuple(new)
        acc = lax.fori_loop(0, KV // UF, exp_it,
                            tuple(jnp.zeros((16,), jnp.float32)
                                  for _ in range(NCTX)))
        winv = [None] * NCTX
        for i, c in enumerate(CC):
            c["sbuf"][...] = plsc.cumsum(acc[i])
            tot = plsc.load_gather(c["sbuf"], [full15])
            winv[i] = 1.0 / tot

        def out_body(v):
            sl = pl.ds(v * 16, 16)
            for i, c in enumerate(CC):
                xbuf[rr[i], sl] = c["ebuf"][sl] * winv[i] * c["pA"][sl]
        vloop(KV, out_body)
        return carry

    def exp_body(e, carry, t0):
        pltpu.sync_copy(gate_hbm.at[e, pl.ds(t0, CHT)], gbuf)
        lax.fori_loop(0, CHT // NCTX, row_group_body, 0)
        pltpu.sync_copy(xbuf, xe_hbm.at[e, pl.ds(t0, CHT)])
        return carry

    def chunk_body(ci, carry):
        t0 = wid * TOK_PER_W + ci * CHT
        pltpu.sync_copy(h_hbm.at[pl.ds(t0, CHT)], hbuf)
        lax.fori_loop(0, NEXP, lambda e, c: exp_body(e, c, t0), 0)
        return carry

    lax.fori_loop(0, NCHUNK, chunk_body, 0)


def _sc_topk(gate_all, h):
    mesh = plsc.VectorSubcoreMesh(core_axis_name="c", subcore_axis_name="s")
    return pl.kernel(
        _sc_body,
        out_type=jax.ShapeDtypeStruct((NEXP, T, K), jnp.float32),
        mesh=mesh,
        compiler_params=pltpu.CompilerParams(needs_layout_passes=False),
        scratch_types=[
            pltpu.VMEM((CHT, EMBED), jnp.float32),
            pltpu.VMEM((CHT, EMBED), jnp.float32),
            pltpu.VMEM((CHT, K), jnp.float32),
        ] + NCTX * [
            pltpu.VMEM((EMBED,), jnp.int32),
            pltpu.VMEM((EMBED,), jnp.int32),
            pltpu.VMEM((EMBED,), jnp.float32),
            pltpu.VMEM((EMBED,), jnp.float32),
            pltpu.VMEM((K,), jnp.float32),
            pltpu.VMEM((1024,), jnp.int32),
            pltpu.VMEM((256,), jnp.int32),
            pltpu.VMEM((16,), jnp.float32),
        ],
    )(gate_all, h)


def kernel(hidden_states, posembed, W_pos, b_pos, W_gate, b_gate,
           Wg_e, Wu_e, Wd_e, Wg_s, Wu_s, Wd_s,
           ln_g, ln_b, W1, b1, W2, b2):
    h = hidden_states[0]
    b_pos2 = b_pos[None]
    b_gate2 = b_gate[None]

    pe = pl.pallas_call(
        _pe_body,
        out_shape=jax.ShapeDtypeStruct((NEXP, EMBED), jnp.float32),
    )(posembed, W_pos, b_pos2)

    grid = (T // TB,)
    full2 = lambda a, b: pl.BlockSpec((a, b), lambda i: (0, 0))
    tile2 = lambda w: pl.BlockSpec((TB, w), lambda i: (i, 0))

    gate_all = pl.pallas_call(
        _gate_body,
        grid=grid,
        in_specs=[tile2(EMBED), full2(NEXP, EMBED), full2(EMBED, EMBED),
                  full2(1, EMBED)],
        out_specs=pl.BlockSpec((NEXP, TB, EMBED), lambda i: (0, i, 0)),
        out_shape=jax.ShapeDtypeStruct((NEXP, T, EMBED), jnp.float32),
    )(h, pe, W_gate, b_gate2)

    xe = _sc_topk(gate_all, h)

    y0 = pl.pallas_call(
        _shared_body,
        grid=grid,
        in_specs=[tile2(EMBED), full2(INTER_S, EMBED), full2(INTER_S, EMBED),
                  full2(EMBED, INTER_S)],
        out_specs=tile2(EMBED),
        out_shape=jax.ShapeDtypeStruct((T, EMBED), jnp.float32),
    )(h, Wg_s, Wu_s, Wd_s)

    full3 = lambda s: pl.BlockSpec(s, lambda i: (0, 0, 0))
    out = pl.pallas_call(
        _post_body,
        grid=grid,
        in_specs=[pl.BlockSpec((NEXP, TB, K), lambda i: (0, i, 0)),
                  tile2(EMBED),
                  full3((NEXP, EMBED, K)), full3((NEXP, EMBED, K)),
                  full3((NEXP, EMBED, EMBED)),
                  full2(1, EMBED), full2(1, EMBED),
                  full2(EMBED, EMBED), full2(1, EMBED),
                  full2(EMBED, EMBED), full2(1, EMBED)],
        out_specs=tile2(EMBED),
        out_shape=jax.ShapeDtypeStruct((T, EMBED), jnp.float32),
    )(xe, y0, Wg_e, Wu_e, Wd_e, ln_g[None], ln_b[None],
      W1, b1[None], W2, b2[None])
    return out[None]


# speedup vs baseline: 1.6265x; 1.0008x over previous
import jax
import jax.numpy as jnp
from jax import lax
from jax.experimental import pallas as pl
from jax.experimental.pallas import tpu as pltpu, tpu_sc as plsc

EMBED = 768
NEXP = 8
K = 384
INTER_S = 1536
T = 2048
TB = 256
NW = 32
TOK_PER_W = T // NW
CHT = 8
NCHUNK = TOK_PER_W // CHT
NV = EMBED // 16
KV = K // 16
MASK31 = 0x7FFFFFFF

_CONTRACT_MINOR = (((1,), (1,)), ((), ()))


def _dotT(a, b):
    return lax.dot_general(a, b, _CONTRACT_MINOR,
                           preferred_element_type=jnp.float32)


def _pe_body(pos_ref, Wp_ref, bp_ref, pe_ref):
    logits = _dotT(pos_ref[...], Wp_ref[...]) + bp_ref[...]
    z = logits - jnp.max(logits, axis=-1, keepdims=True)
    e = jnp.exp(z)
    pe_ref[...] = e / jnp.sum(e, axis=-1, keepdims=True)


def _gate_body(h_ref, pe_ref, Wg_ref, bg_ref, gate_ref):
    h = h_ref[...]
    for e in range(NEXP):
        gate_ref[e] = _dotT(h * pe_ref[e], Wg_ref[...]) + bg_ref[...]


def _shared_body(h_ref, Wgs_ref, Wus_ref, Wds_ref, y0_ref):
    h = h_ref[...]
    m = jax.nn.silu(_dotT(h, Wgs_ref[...])) * _dotT(h, Wus_ref[...])
    y0_ref[...] = _dotT(m, Wds_ref[...])


def _post_body(xe_ref, y0_ref, Wge_ref, Wue_ref, Wde_ref,
               lng_ref, lnb_ref, W1_ref, b1_ref, W2_ref, b2_ref, o_ref):
    y = y0_ref[...]
    for e in range(NEXP):
        x = xe_ref[e]
        a = _dotT(x, Wge_ref[e])
        b = _dotT(x, Wue_ref[e])
        y = y + _dotT(jax.nn.silu(a) * b, Wde_ref[e])
    mean = jnp.mean(y, axis=-1, keepdims=True)
    var = jnp.mean((y - mean) ** 2, axis=-1, keepdims=True)
    y = (y - mean) * lax.rsqrt(var + 1e-6) * lng_ref[...] + lnb_ref[...]
    z = jax.nn.silu(_dotT(y, W1_ref[...]) + b1_ref[...])
    o_ref[...] = _dotT(z, W2_ref[...]) + b2_ref[...]


def _desc_key(f):
    u = plsc.bitcast(f, jnp.int32)
    s = lax.shift_right_arithmetic(u, 31)
    return jnp.bitwise_xor(u, jnp.bitwise_and(jnp.bitwise_not(s), MASK31))


def _inv_desc_key(k):
    s = lax.shift_right_arithmetic(k, 31)
    u = jnp.bitwise_xor(k, jnp.bitwise_and(jnp.bitwise_not(s), MASK31))
    return plsc.bitcast(u, jnp.float32)


def _digit(k, p):
    if p:
        k = lax.shift_right_logical(k, 8 * p)
    return jnp.bitwise_and(k, 255)


NCTX = 8
UF = 4
_CTX_BUFS = ("kA", "kB", "pA", "pB", "ebuf", "h4", "offs", "sbuf")


def _sc_body(gate_hbm, h_hbm, xe_hbm, hbuf, gbuf, xbuf, *ctx_bufs):
    wid = lax.axis_index("s") * 2 + lax.axis_index("c")
    zero16 = jnp.zeros((16,), jnp.int32)
    full15 = jnp.full((16,), 15, jnp.int32)
    nb = len(_CTX_BUFS)
    CC = tuple(dict(zip(_CTX_BUFS, ctx_bufs[i * nb:(i + 1) * nb]))
               for i in range(NCTX))

    def scan_offsets(p):
        run = [jnp.int32(-1)] * NCTX
        for v in range(16):
            sl = pl.ds(v * 16, 16)
            slh = pl.ds(p * 256 + v * 16, 16)
            for i, c in enumerate(CC):
                hv = c["h4"][slh]
                inc = plsc.cumsum(hv)
                c["offs"][sl] = inc - hv + run[i]
                run[i] = run[i] + jnp.sum(hv)

    def vloop(n_vregs, body):
        def it_body(it, carry):
            for j in range(UF):
                body(it * UF + j)
            return carry
        lax.fori_loop(0, n_vregs // UF, it_body, 0)

    def radix_pass(p, rr):
        ks = "kA" if p % 2 == 0 else "kB"
        kd = "kB" if p % 2 == 0 else "kA"
        pd = "pB" if p % 2 == 0 else "pA"

        def psrc(c, i, sl):
            if p == 0:
                return hbuf[rr[i], sl]
            return c["pB" if p % 2 == 1 else "pA"][sl]

        scan_offsets(p)

        def body(v):
            sl = pl.ds(v * 16, 16)
            for i, c in enumerate(CC):
                k = c[ks][sl]
                d = _digit(k, p)
                cnt, lastm = plsc.scan_count(d)
                base = plsc.load_gather(c["offs"], [d])
                pos = base + cnt
                plsc.store_scatter(c[kd], [pos], k)
                plsc.store_scatter(c[pd], [pos], psrc(c, i, sl))
                plsc.addupdate_scatter(c["offs"], [d], cnt, mask=lastm)
        vloop(NV, body)

    def row_group_body(rp, carry):
        rr = tuple(NCTX * rp + i for i in range(NCTX))

        def zero_body(v):
            sl = pl.ds(v * 16, 16)
            for c in CC:
                c["h4"][sl] = zero16
        vloop(64, zero_body)

        def hist_body(v):
            sl = pl.ds(v * 16, 16)
            for i, c in enumerate(CC):
                k = _desc_key(gbuf[rr[i], sl])
                c["kA"][sl] = k
                d0 = _digit(k, 0)
                d1 = _digit(k, 1)
                d2 = _digit(k, 2)
                d3 = _digit(k, 3)
                cnt0, last0 = plsc.scan_count(d0)
                cnt1, last1 = plsc.scan_count(d1)
                cnt2, last2 = plsc.scan_count(d2)
                cnt3, last3 = plsc.scan_count(d3)
                plsc.addupdate_scatter(c["h4"], [d0], cnt0, mask=last0)
                plsc.addupdate_scatter(c["h4"], [d1 + 256], cnt1, mask=last1)
                plsc.addupdate_scatter(c["h4"], [d2 + 512], cnt2, mask=last2)
                plsc.addupdate_scatter(c["h4"], [d3 + 768], cnt3, mask=last3)
        vloop(NV, hist_body)

        for p in range(4):
            radix_pass(p, rr)
        mv = [None] * NCTX
        for i, c in enumerate(CC):
            v0 = _inv_desc_key(c["kA"][pl.ds(0, 16)])
            c["sbuf"][...] = plsc.cummax(v0)
            mv[i] = plsc.load_gather(c["sbuf"], [full15])

        def exp_it(it, accs):
            new = list(accs)
            for j in range(UF):
                sl = pl.ds((it * UF + j) * 16, 16)
                for i, c in enumerate(CC):
                    ej = jnp.exp(_inv_desc_key(c["kA"][sl]) - mv[i])
                    c["ebuf"][sl] = ej
                    new[i] = new[i] + ej
            return tuple(new)
        acc = lax.fori_loop(0, KV // UF, exp_it,
                            tuple(jnp.zeros((16,), jnp.float32)
                                  for _ in range(NCTX)))
        winv = [None] * NCTX
        for i, c in enumerate(CC):
            c["sbuf"][...] = plsc.cumsum(acc[i])
            tot = plsc.load_gather(c["sbuf"], [full15])
            winv[i] = 1.0 / tot

        def out_body(v):
            sl = pl.ds(v * 16, 16)
            for i, c in enumerate(CC):
                xbuf[rr[i], sl] = c["ebuf"][sl] * winv[i] * c["pA"][sl]
        vloop(KV, out_body)
        return carry

    def exp_body(e, carry, t0):
        pltpu.sync_copy(gate_hbm.at[e, pl.ds(t0, CHT)], gbuf)
        lax.fori_loop(0, CHT // NCTX, row_group_body, 0)
        pltpu.sync_copy(xbuf, xe_hbm.at[e, pl.ds(t0, CHT)])
        return carry

    def chunk_body(ci, carry):
        t0 = wid * TOK_PER_W + ci * CHT
        pltpu.sync_copy(h_hbm.at[pl.ds(t0, CHT)], hbuf)
        lax.fori_loop(0, NEXP, lambda e, c: exp_body(e, c, t0), 0)
        return carry

    lax.fori_loop(0, NCHUNK, chunk_body, 0)


def _sc_topk(gate_all, h):
    mesh = plsc.VectorSubcoreMesh(core_axis_name="c", subcore_axis_name="s")
    return pl.kernel(
        _sc_body,
        out_type=jax.ShapeDtypeStruct((NEXP, T, K), jnp.float32),
        mesh=mesh,
        compiler_params=pltpu.CompilerParams(needs_layout_passes=False),
        scratch_types=[
            pltpu.VMEM((CHT, EMBED), jnp.float32),
            pltpu.VMEM((CHT, EMBED), jnp.float32),
            pltpu.VMEM((CHT, K), jnp.float32),
        ] + NCTX * [
            pltpu.VMEM((EMBED,), jnp.int32),
            pltpu.VMEM((EMBED,), jnp.int32),
            pltpu.VMEM((EMBED,), jnp.float32),
            pltpu.VMEM((EMBED,), jnp.float32),
            pltpu.VMEM((K,), jnp.float32),
            pltpu.VMEM((1024,), jnp.int32),
            pltpu.VMEM((256,), jnp.int32),
            pltpu.VMEM((16,), jnp.float32),
        ],
    )(gate_all, h)


def kernel(hidden_states, posembed, W_pos, b_pos, W_gate, b_gate,
           Wg_e, Wu_e, Wd_e, Wg_s, Wu_s, Wd_s,
           ln_g, ln_b, W1, b1, W2, b2):
    h = hidden_states[0]
    b_pos2 = b_pos[None]
    b_gate2 = b_gate[None]

    pe = pl.pallas_call(
        _pe_body,
        out_shape=jax.ShapeDtypeStruct((NEXP, EMBED), jnp.float32),
    )(posembed, W_pos, b_pos2)

    grid = (T // TB,)
    full2 = lambda a, b: pl.BlockSpec((a, b), lambda i: (0, 0))
    tile2 = lambda w: pl.BlockSpec((TB, w), lambda i: (i, 0))

    gate_all = pl.pallas_call(
        _gate_body,
        grid=grid,
        in_specs=[tile2(EMBED), full2(NEXP, EMBED), full2(EMBED, EMBED),
                  full2(1, EMBED)],
        out_specs=pl.BlockSpec((NEXP, TB, EMBED), lambda i: (0, i, 0)),
        out_shape=jax.ShapeDtypeStruct((NEXP, T, EMBED), jnp.float32),
    )(h, pe, W_gate, b_gate2)

    xe = _sc_topk(gate_all, h)

    y0 = pl.pallas_call(
        _shared_body,
        grid=grid,
        in_specs=[tile2(EMBED), full2(INTER_S, EMBED), full2(INTER_S, EMBED),
                  full2(EMBED, INTER_S)],
        out_specs=tile2(EMBED),
        out_shape=jax.ShapeDtypeStruct((T, EMBED), jnp.float32),
    )(h, Wg_s, Wu_s, Wd_s)

    full3 = lambda s: pl.BlockSpec(s, lambda i: (0, 0, 0))
    out = pl.pallas_call(
        _post_body,
        grid=grid,
        in_specs=[pl.BlockSpec((NEXP, TB, K), lambda i: (0, i, 0)),
                  tile2(EMBED),
                  full3((NEXP, EMBED, K)), full3((NEXP, EMBED, K)),
                  full3((NEXP, EMBED, EMBED)),
                  full2(1, EMBED), full2(1, EMBED),
                  full2(EMBED, EMBED), full2(1, EMBED),
                  full2(EMBED, EMBED), full2(1, EMBED)],
        out_specs=tile2(EMBED),
        out_shape=jax.ShapeDtypeStruct((T, EMBED), jnp.float32),
    )(xe, y0, Wg_e, Wu_e, Wd_e, ln_g[None], ln_b[None],
      W1, b1[None], W2, b2[None])
    return out[None]
